# baseline (speedup 1.0000x reference)
import functools

import jax
import jax.numpy as jnp
from jax import lax
from jax.experimental import pallas as pl
from jax.experimental.pallas import tpu as pltpu
from jax.experimental.pallas import tpu_sc as plsc

BATCH = 16384
EMBED = 32
NUM_CORES = 2
NUM_SUBCORES = 16
LANES = 16
NUM_WORKERS = NUM_CORES * NUM_SUBCORES
BPW = BATCH // NUM_WORKERS
IDX_CHUNK = 128
N_CHUNKS = BPW // IDX_CHUNK

_mesh = plsc.VectorSubcoreMesh(core_axis_name="c", subcore_axis_name="s")


@functools.partial(
    pl.kernel,
    out_type=jax.ShapeDtypeStruct((BATCH,), jnp.float32),
    mesh=_mesh,
    compiler_params=pltpu.CompilerParams(needs_layout_passes=False,
                                         use_tc_tiling_on_sc=False),
    scratch_types=[
        pltpu.VMEM((N_CHUNKS, IDX_CHUNK), jnp.int32),
        pltpu.VMEM((N_CHUNKS, IDX_CHUNK), jnp.int32),
        pltpu.VMEM((BPW, EMBED), jnp.float32),
        pltpu.VMEM((BPW, EMBED), jnp.float32),
        pltpu.VMEM((BPW,), jnp.float32),
        pltpu.SemaphoreType.DMA,
    ],
)
def _mf_dot_kernel(xu_hbm, xi_hbm, w_hbm, h_hbm, out_hbm,
                   uidx, vidx, urows, vrows, outv, sem):
    wid = lax.axis_index("s") * NUM_CORES + lax.axis_index("c")
    base = wid * BPW
    lanes = lax.iota(jnp.int32, LANES)

    for c in range(N_CHUNKS):
        pltpu.sync_copy(xu_hbm.at[pl.ds(base + c * IDX_CHUNK, IDX_CHUNK)],
                        uidx.at[c])
        pltpu.sync_copy(xi_hbm.at[pl.ds(base + c * IDX_CHUNK, IDX_CHUNK)],
                        vidx.at[c])

    copies = []
    for c in range(N_CHUNKS):
        copies.append(pltpu.async_copy(
            w_hbm.at[uidx.at[c]],
            urows.at[pl.ds(c * IDX_CHUNK, IDX_CHUNK), :], sem))
        copies.append(pltpu.async_copy(
            h_hbm.at[vidx.at[c]],
            vrows.at[pl.ds(c * IDX_CHUNK, IDX_CHUNK), :], sem))
    for cp in copies:
        cp.wait()

    def block_body(blk, _):
        r0 = blk * LANES
        acc = jnp.zeros((LANES,), jnp.float32)
        for j in range(LANES):
            i = r0 + j
            u0 = urows[i, pl.ds(0, LANES)]
            u1 = urows[i, pl.ds(LANES, LANES)]
            v0 = vrows[i, pl.ds(0, LANES)]
            v1 = vrows[i, pl.ds(LANES, LANES)]
            s = jnp.sum(u0 * v0 + u1 * v1)
            acc = jnp.where(lanes == j, s, acc)
        outv[pl.ds(r0, LANES)] = acc
        return 0

    lax.fori_loop(0, BPW // LANES, block_body, 0)

    pltpu.sync_copy(outv, out_hbm.at[pl.ds(base, BPW)])


def kernel(x, W, H):
    return _mf_dot_kernel(x[:, 0], x[:, 1], W, H)


# speedup vs baseline: 2.8573x; 2.8573x over previous
import functools

import jax
import jax.numpy as jnp
from jax import lax
from jax.experimental import pallas as pl
from jax.experimental.pallas import tpu as pltpu
from jax.experimental.pallas import tpu_sc as plsc

BATCH = 16384
EMBED = 32
NUM_CORES = 2
NUM_SUBCORES = 16
LANES = 16
NUM_WORKERS = NUM_CORES * NUM_SUBCORES
BPW = BATCH // NUM_WORKERS
NBUF = 2

_mesh = plsc.VectorSubcoreMesh(core_axis_name="c", subcore_axis_name="s")


@functools.partial(
    pl.kernel,
    out_type=jax.ShapeDtypeStruct((BATCH,), jnp.float32),
    mesh=_mesh,
    compiler_params=pltpu.CompilerParams(needs_layout_passes=False,
                                         use_tc_tiling_on_sc=True),
    scratch_types=[
        pltpu.VMEM((BPW + LANES,), jnp.int32),
        pltpu.VMEM((BPW + LANES,), jnp.int32),
        pltpu.VMEM((NBUF, EMBED, 128), jnp.float32),
        pltpu.VMEM((NBUF, EMBED, 128), jnp.float32),
        pltpu.VMEM((BPW,), jnp.float32),
        pltpu.SemaphoreType.DMA,
        pltpu.SemaphoreType.DMA,
        pltpu.SemaphoreType.DMA,
        pltpu.SemaphoreType.DMA,
    ],
)
def _mf_dot_kernel(xu_hbm, xi_hbm, wt_hbm, ht_hbm, out_hbm,
                   uidx, vidx, wtile, htile, outv, sw0, sw1, sh0, sh1):
    wid = lax.axis_index("s") * NUM_CORES + lax.axis_index("c")
    base = wid * BPW
    lanes = lax.iota(jnp.int32, LANES)
    semw = (sw0, sw1)
    semh = (sh0, sh1)

    pltpu.sync_copy(xu_hbm.at[pl.ds(base, BPW)], uidx.at[pl.ds(0, BPW)])
    pltpu.sync_copy(xi_hbm.at[pl.ds(base, BPW)], vidx.at[pl.ds(0, BPW)])
    uidx[pl.ds(BPW, LANES)] = jnp.zeros((LANES,), jnp.int32)
    vidx[pl.ds(BPW, LANES)] = jnp.zeros((LANES,), jnp.int32)

    def issue(u, v, slot):
        cu = pl.multiple_of((u >> 7) << 7, 128)
        cv = pl.multiple_of((v >> 7) << 7, 128)
        cw = pltpu.async_copy(wt_hbm.at[:, pl.ds(cu, 128)], wtile.at[slot],
                              semw[slot])
        ch = pltpu.async_copy(ht_hbm.at[:, pl.ds(cv, 128)], htile.at[slot],
                              semh[slot])
        return cw, ch

    uvec0 = uidx[pl.ds(0, LANES)]
    vvec0 = vidx[pl.ds(0, LANES)]
    issue(uvec0[0], vvec0[0], 0)

    def group_body(g, carry):
        uvec, vvec = carry
        acc = jnp.zeros((LANES,), jnp.float32)
        unext, vnext = uvec, vvec
        for j in range(LANES):
            if j == LANES - 1:
                unext = uidx[pl.ds((g + 1) * LANES, LANES)]
                vnext = vidx[pl.ds((g + 1) * LANES, LANES)]
                nu, nv = unext[0], vnext[0]
            else:
                nu, nv = uvec[j + 1], vvec[j + 1]
            slot = j % NBUF
            nslot = (j + 1) % NBUF
            issue(nu, nv, nslot)
            pltpu.make_async_copy(wt_hbm.at[:, pl.ds(0, 128)],
                                  wtile.at[slot], semw[slot]).wait()
            pltpu.make_async_copy(ht_hbm.at[:, pl.ds(0, 128)],
                                  htile.at[slot], semh[slot]).wait()
            lu = jnp.broadcast_to(uvec[j] & 127, (LANES,))
            lv = jnp.broadcast_to(vvec[j] & 127, (LANES,))
            w0 = plsc.load_gather(wtile.at[slot], [lanes, lu])
            w1 = plsc.load_gather(wtile.at[slot], [lanes + LANES, lu])
            h0 = plsc.load_gather(htile.at[slot], [lanes, lv])
            h1 = plsc.load_gather(htile.at[slot], [lanes + LANES, lv])
            s = jnp.sum(w0 * h0 + w1 * h1)
            acc = jnp.where(lanes == j, s, acc)
        outv[pl.ds(g * LANES, LANES)] = acc
        return unext, vnext

    lax.fori_loop(0, BPW // LANES, group_body, (uvec0, vvec0))
    pltpu.make_async_copy(wt_hbm.at[:, pl.ds(0, 128)], wtile.at[0],
                          semw[0]).wait()
    pltpu.make_async_copy(ht_hbm.at[:, pl.ds(0, 128)], htile.at[0],
                          semh[0]).wait()

    pltpu.sync_copy(outv, out_hbm.at[pl.ds(base, BPW)])


def kernel(x, W, H):
    return _mf_dot_kernel(x[:, 0], x[:, 1], W.T, H.T)


# speedup vs baseline: 3.8075x; 1.3325x over previous
import functools

import jax
import jax.numpy as jnp
from jax import lax
from jax.experimental import pallas as pl
from jax.experimental.pallas import tpu as pltpu
from jax.experimental.pallas import tpu_sc as plsc

BATCH = 16384
EMBED = 32
NUM_CORES = 2
NUM_SUBCORES = 16
LANES = 16
NUM_WORKERS = NUM_CORES * NUM_SUBCORES
BPW = BATCH // NUM_WORKERS
NBUF = 4
AHEAD = NBUF - 1

_mesh = plsc.VectorSubcoreMesh(core_axis_name="c", subcore_axis_name="s")


@functools.partial(
    pl.kernel,
    out_type=jax.ShapeDtypeStruct((BATCH,), jnp.float32),
    mesh=_mesh,
    compiler_params=pltpu.CompilerParams(needs_layout_passes=False,
                                         use_tc_tiling_on_sc=True),
    scratch_types=[
        pltpu.VMEM((BPW + LANES,), jnp.int32),
        pltpu.VMEM((BPW + LANES,), jnp.int32),
        pltpu.VMEM((NBUF, EMBED, 128), jnp.float32),
        pltpu.VMEM((NBUF, EMBED, 128), jnp.float32),
        pltpu.VMEM((BPW,), jnp.float32),
    ] + [pltpu.SemaphoreType.DMA] * (2 * NBUF),
)
def _mf_dot_kernel(xu_hbm, xi_hbm, wt_hbm, ht_hbm, out_hbm,
                   uidx, vidx, wtile, htile, outv, *sems):
    semw = sems[:NBUF]
    semh = sems[NBUF:]
    wid = lax.axis_index("s") * NUM_CORES + lax.axis_index("c")
    base = wid * BPW
    lanes = lax.iota(jnp.int32, LANES)

    pltpu.sync_copy(xu_hbm.at[pl.ds(base, BPW)], uidx.at[pl.ds(0, BPW)])
    pltpu.sync_copy(xi_hbm.at[pl.ds(base, BPW)], vidx.at[pl.ds(0, BPW)])
    uidx[pl.ds(BPW, LANES)] = jnp.zeros((LANES,), jnp.int32)
    vidx[pl.ds(BPW, LANES)] = jnp.zeros((LANES,), jnp.int32)

    def issue(u, v, slot):
        cu = pl.multiple_of((u >> 7) << 7, 128)
        cv = pl.multiple_of((v >> 7) << 7, 128)
        pltpu.async_copy(wt_hbm.at[:, pl.ds(cu, 128)], wtile.at[slot],
                         semw[slot])
        pltpu.async_copy(ht_hbm.at[:, pl.ds(cv, 128)], htile.at[slot],
                         semh[slot])

    def drain(slot):
        pltpu.make_async_copy(wt_hbm.at[:, pl.ds(0, 128)],
                              wtile.at[slot], semw[slot]).wait()
        pltpu.make_async_copy(ht_hbm.at[:, pl.ds(0, 128)],
                              htile.at[slot], semh[slot]).wait()

    uvec0 = uidx[pl.ds(0, LANES)]
    vvec0 = vidx[pl.ds(0, LANES)]
    for p in range(AHEAD):
        issue(uvec0[p], vvec0[p], p)

    def group_body(g, carry):
        uvec, vvec = carry
        acc = jnp.zeros((LANES,), jnp.float32)
        unext, vnext = uvec, vvec
        for j in range(LANES):
            if j == LANES - AHEAD:
                unext = uidx[pl.ds((g + 1) * LANES, LANES)]
                vnext = vidx[pl.ds((g + 1) * LANES, LANES)]
            p = j + AHEAD
            if p < LANES:
                nu, nv = uvec[p], vvec[p]
            else:
                nu, nv = unext[p - LANES], vnext[p - LANES]
            slot = j % NBUF
            issue(nu, nv, p % NBUF)
            drain(slot)
            lu = jnp.broadcast_to(uvec[j] & 127, (LANES,))
            lv = jnp.broadcast_to(vvec[j] & 127, (LANES,))
            w0 = plsc.load_gather(wtile.at[slot], [lanes, lu])
            w1 = plsc.load_gather(wtile.at[slot], [lanes + LANES, lu])
            h0 = plsc.load_gather(htile.at[slot], [lanes, lv])
            h1 = plsc.load_gather(htile.at[slot], [lanes + LANES, lv])
            s = jnp.sum(w0 * h0 + w1 * h1)
            acc = jnp.where(lanes == j, s, acc)
        outv[pl.ds(g * LANES, LANES)] = acc
        return unext, vnext

    lax.fori_loop(0, BPW // LANES, group_body, (uvec0, vvec0))
    for p in range(AHEAD):
        drain(p % NBUF)

    pltpu.sync_copy(outv, out_hbm.at[pl.ds(base, BPW)])


def kernel(x, W, H):
    return _mf_dot_kernel(x[:, 0], x[:, 1], W.T, H.T)
